# baseline (speedup 1.0000x reference)
import functools

import jax
import jax.numpy as jnp
from jax import lax
from jax.experimental import pallas as pl
from jax.experimental.pallas import tpu as pltpu
from jax.experimental.pallas import tpu_sc as plsc

N_CHECKIN = 100000
N_POI = 20000
HIDDEN = 128
BLK = 1000

C_EDGE = 1920
G = 128
NSUB = 16
BROWS = 10000
BPAD = 10240
E_PAD_BIG = 614400
E_PAD_SP = 337920


def _pad_edges(src, dst, n_dst, e_pad):
    e = src.shape[0]
    pad = e_pad - e
    srcp = jnp.concatenate([src, jnp.zeros((pad,), jnp.int32)])
    dstp = jnp.concatenate([dst, jnp.full((pad,), n_dst, jnp.int32)])
    return srcp, dstp


def _make_agg(n_src, n_dst, e_pad):
    nblk = n_dst // (2 * BROWS)
    et = e_pad // NSUB
    nch = et // C_EDGE
    nsc = C_EDGE // G
    mesh = plsc.VectorSubcoreMesh(core_axis_name="c", subcore_axis_name="s")

    @functools.partial(
        pl.kernel,
        out_type=jax.ShapeDtypeStruct((n_dst, HIDDEN), jnp.float32),
        mesh=mesh,
        compiler_params=pltpu.CompilerParams(needs_layout_passes=False),
        scratch_types=[
            pltpu.VMEM((C_EDGE,), jnp.int32),
            pltpu.VMEM((C_EDGE,), jnp.int32),
            pltpu.VMEM((C_EDGE + 16,), jnp.int32),
            pltpu.VMEM((C_EDGE + 16,), jnp.int32),
            pltpu.VMEM((nsc, G), jnp.int32),
            pltpu.VMEM((G, HIDDEN), jnp.float32),
            pltpu.VMEM((G, HIDDEN), jnp.float32),
            pltpu.VMEM_SHARED((BPAD, HIDDEN), jnp.float32),
            pltpu.SemaphoreType.DMA,
            pltpu.SemaphoreType.DMA,
        ],
    )
    def agg(table, srcp, dstp, out,
            dch, sch, s1d, d1d, d2d, rows0, rows1, block, gsem0, gsem1):
        cid = lax.axis_index("c")
        sid = lax.axis_index("s")
        ebase = sid * et
        zero16f = jnp.zeros((16,), jnp.float32)
        zero16i = jnp.zeros((16,), jnp.int32)
        dum16 = jnp.full((16,), BROWS, jnp.int32)

        def zs_body(i, _):
            s1d[pl.ds(i * 16, 16)] = zero16i
            return 0
        lax.fori_loop(0, C_EDGE // 16, zs_body, 0)

        for blk in range(nblk):
            r0 = (cid * nblk + blk) * BROWS

            def zb_body(i, _):
                for k in range(HIDDEN // 16):
                    rows0[i, pl.ds(k * 16, 16)] = zero16f
                return 0
            lax.fori_loop(0, G, zb_body, 0)
            for k in range(BPAD // NSUB // G):
                pltpu.sync_copy(rows0, block.at[pl.ds(sid * (BPAD // NSUB)
                                                      + k * G, G)])
            plsc.subcore_barrier()

            def chunk_body(ch, _):
                base = ebase + ch * C_EDGE
                pltpu.sync_copy(dstp.at[pl.ds(base, C_EDGE)], dch)
                pltpu.sync_copy(srcp.at[pl.ds(base, C_EDGE)], sch)

                def zd_body(i, _):
                    d1d[pl.ds(i * 16, 16)] = dum16
                    return 0
                lax.fori_loop(0, C_EDGE // 16, zd_body, 0)

                def compact(i, cnt):
                    vd = dch[pl.ds(i * 16, 16)]
                    vs = sch[pl.ds(i * 16, 16)]
                    m = jnp.logical_and(vd >= r0, vd < r0 + BROWS)
                    cum = plsc.cumsum(m.astype(jnp.int32))
                    pos = jnp.where(m, cum - 1 + cnt, C_EDGE)
                    plsc.store_scatter(s1d, [pos], vs)
                    plsc.store_scatter(d1d, [pos], vd - r0)
                    return cnt + plsc.all_reduce_population_count(m)[0]
                cnt = lax.fori_loop(0, C_EDGE // 16, compact, jnp.int32(0))

                bufs = ((rows0, gsem0), (rows1, gsem1))
                descs = {}

                def fire_gather(j):
                    buf, sem = bufs[j % 2]
                    for k in range(G // 16):
                        d2d[j, pl.ds(k * 16, 16)] = \
                            d1d[pl.ds(j * G + k * 16, 16)]
                    descs[j] = pltpu.async_copy(
                        table.at[s1d.at[pl.ds(j * G, G)]], buf, sem)

                @pl.when(cnt > 0)
                def _():
                    fire_gather(0)

                for j in range(nsc):
                    @pl.when(j * G < cnt)
                    def _(j=j):
                        if j + 1 < nsc:
                            @pl.when((j + 1) * G < cnt)
                            def _():
                                fire_gather(j + 1)
                        descs[j].wait()
                        pltpu.sync_copy(bufs[j % 2][0],
                                        block.at[d2d.at[j]], add=True)
                return 0
            lax.fori_loop(0, nch, chunk_body, 0)
            plsc.subcore_barrier()

            w = 80
            nchunks_wb = BROWS // w
            for k in range((nchunks_wb + NSUB - 1) // NSUB):
                idx = sid + k * NSUB

                @pl.when(idx < nchunks_wb)
                def _():
                    off = pl.multiple_of(idx * w, w)
                    pltpu.sync_copy(block.at[pl.ds(off, w)],
                                    rows0.at[pl.ds(0, w)])
                    pltpu.sync_copy(rows0.at[pl.ds(0, w)],
                                    out.at[pl.ds(r0 + off, w)])
            plsc.subcore_barrier()

    return agg


_EDGE_DEFS = (
    ("seq", N_CHECKIN, E_PAD_BIG),
    ("vtd", N_CHECKIN, E_PAD_BIG),
    ("vis", N_POI, E_PAD_BIG),
    ("sp", N_POI, E_PAD_SP),
)


def _make_counts():
    mesh = plsc.VectorSubcoreMesh(core_axis_name="c", subcore_axis_name="s")
    allocs = {N_CHECKIN: 50176, N_POI: 10240}

    @functools.partial(
        pl.kernel,
        out_type=[jax.ShapeDtypeStruct((n, ), jnp.float32)
                  for _, n, _ in _EDGE_DEFS],
        mesh=mesh,
        compiler_params=pltpu.CompilerParams(needs_layout_passes=False),
        scratch_types=[
            pltpu.VMEM((C_EDGE,), jnp.int32),
            pltpu.VMEM((C_EDGE + 16,), jnp.int32),
            pltpu.VMEM((C_EDGE // G, G), jnp.int32),
            pltpu.VMEM((G,), jnp.float32),
            pltpu.VMEM((50176 // NSUB,), jnp.float32),
            pltpu.VMEM((1000,), jnp.float32),
            pltpu.VMEM_SHARED((50176,), jnp.float32),
            pltpu.VMEM_SHARED((50176,), jnp.float32),
            pltpu.VMEM_SHARED((10240,), jnp.float32),
            pltpu.VMEM_SHARED((10240,), jnp.float32),
        ],
    )
    def counts(d_seq, d_vtd, d_vis, d_sp,
               o_seq, o_vtd, o_vis, o_sp,
               dch, d1d, d2d, ones, zbuf, wbuf, c0, c1, c2, c3):
        cid = lax.axis_index("c")
        sid = lax.axis_index("s")
        one16 = jnp.ones((16,), jnp.float32)
        zero16f = jnp.zeros((16,), jnp.float32)

        def zo_body(i, _):
            ones[pl.ds(i * 16, 16)] = one16
            return 0
        lax.fori_loop(0, G // 16, zo_body, 0)

        def zz_body(i, _):
            zbuf[pl.ds(i * 16, 16)] = zero16f
            return 0
        lax.fori_loop(0, 50176 // NSUB // 16, zz_body, 0)

        for (nm, n_dst, e_pad), dst_in, out_ref, cspm in zip(
                _EDGE_DEFS, (d_seq, d_vtd, d_vis, d_sp),
                (o_seq, o_vtd, o_vis, o_sp), (c0, c1, c2, c3)):
            nhalf = n_dst // 2
            alloc = allocs[n_dst]
            share = alloc // NSUB
            et = e_pad // NSUB
            nch = et // C_EDGE
            lo = cid * nhalf
            dum16 = jnp.full((16,), nhalf, jnp.int32)

            pltpu.sync_copy(zbuf.at[pl.ds(0, share)],
                            cspm.at[pl.ds(sid * share, share)])
            plsc.subcore_barrier()

            def chunk_body(ch, _):
                base = sid * et + ch * C_EDGE
                pltpu.sync_copy(dst_in.at[pl.ds(base, C_EDGE)], dch)

                def zd_body(i, _):
                    d1d[pl.ds(i * 16, 16)] = dum16
                    return 0
                lax.fori_loop(0, C_EDGE // 16, zd_body, 0)

                def compact(i, cnt):
                    vd = dch[pl.ds(i * 16, 16)] - lo
                    m = jnp.logical_and(vd >= 0, vd < nhalf)
                    cum = plsc.cumsum(m.astype(jnp.int32))
                    pos = jnp.where(m, cum - 1 + cnt, C_EDGE)
                    plsc.store_scatter(d1d, [pos], vd)
                    return cnt + plsc.all_reduce_population_count(m)[0]
                cnt = lax.fori_loop(0, C_EDGE // 16, compact, jnp.int32(0))

                def fire(j, _):
                    @pl.when(j * G < cnt)
                    def _():
                        for k in range(G // 16):
                            d2d[j, pl.ds(k * 16, 16)] = \
                                d1d[pl.ds(j * G + k * 16, 16)]
                        pltpu.sync_copy(ones, cspm.at[d2d.at[j]], add=True)
                    return 0
                lax.fori_loop(0, C_EDGE // G, fire, 0)
                return 0
            lax.fori_loop(0, nch, chunk_body, 0)
            plsc.subcore_barrier()

            nwb = nhalf // 1000
            for k in range((nwb + NSUB - 1) // NSUB):
                idx = sid + k * NSUB

                @pl.when(idx < nwb)
                def _():
                    pltpu.sync_copy(cspm.at[pl.ds(idx * 1000, 1000)], wbuf)
                    pltpu.sync_copy(wbuf, out_ref.at[pl.ds(lo + idx * 1000,
                                                           1000)])
            plsc.subcore_barrier()

    return counts


_agg_cc = _make_agg(N_CHECKIN, N_CHECKIN, E_PAD_BIG)
_agg_pc = _make_agg(N_POI, N_CHECKIN, E_PAD_BIG)
_agg_cp = _make_agg(N_CHECKIN, N_POI, E_PAD_BIG)
_agg_pp = _make_agg(N_POI, N_POI, E_PAD_SP)
_counts_k = _make_counts()


def _transform_body(x_ref, W_ref, b_ref, deg_ref, h_ref, hsc_ref):
    h = jnp.dot(x_ref[:], W_ref[:], preferred_element_type=jnp.float32,
                precision=lax.Precision.HIGHEST) + b_ref[:]
    deg = deg_ref[:]
    dis = jnp.where(deg > 0.0, lax.rsqrt(jnp.maximum(deg, 1e-12)), 0.0)
    h_ref[:] = h
    hsc_ref[:] = dis * h


def _transform(x, W, b, deg, n):
    row = pl.BlockSpec((BLK, HIDDEN), lambda i: (i, 0))
    return pl.pallas_call(
        _transform_body,
        grid=(n // BLK,),
        in_specs=[
            row,
            pl.BlockSpec((HIDDEN, HIDDEN), lambda i: (0, 0)),
            pl.BlockSpec((1, HIDDEN), lambda i: (0, 0)),
            pl.BlockSpec((BLK, 1), lambda i: (i, 0)),
        ],
        out_specs=[row, row],
        out_shape=[jax.ShapeDtypeStruct((n, HIDDEN), jnp.float32)] * 2,
    )(x, W, b.reshape(1, HIDDEN), deg)


def _combine_body(agg1_ref, agg2_ref, h_ref, deg_ref, cnt_ref,
                  W1_ref, W2_ref, W3_ref, b1_ref, b2_ref, pa_ref,
                  c_ref, csc_ref, *, with_prelu, with_scaled):
    deg = deg_ref[:]
    dis = jnp.where(deg > 0.0, lax.rsqrt(jnp.maximum(deg, 1e-12)), 0.0)
    invc = 1.0 / jnp.maximum(cnt_ref[:], 1.0)
    hi = lax.Precision.HIGHEST
    t = dis * jnp.dot(agg1_ref[:], W1_ref[:],
                      preferred_element_type=jnp.float32, precision=hi)
    t = t + b1_ref[:] + b2_ref[:]
    t = t + jnp.dot(invc * agg2_ref[:], W2_ref[:],
                    preferred_element_type=jnp.float32, precision=hi)
    t = t + jnp.dot(h_ref[:], W3_ref[:],
                    preferred_element_type=jnp.float32, precision=hi)
    if with_prelu:
        t = jnp.where(t >= 0.0, t, pa_ref[0, 0] * t)
    c_ref[:] = t
    if with_scaled:
        csc_ref[:] = dis * t


def _combine(agg1, agg2, h, deg, cnt, W1, W2, W3, b1, b2, pa, n,
             with_prelu, with_scaled):
    row = pl.BlockSpec((BLK, HIDDEN), lambda i: (i, 0))
    wspec = pl.BlockSpec((HIDDEN, HIDDEN), lambda i: (0, 0))
    bspec = pl.BlockSpec((1, HIDDEN), lambda i: (0, 0))
    col = pl.BlockSpec((BLK, 1), lambda i: (i, 0))
    nout = 2 if with_scaled else 1
    body = functools.partial(_combine_body, with_prelu=with_prelu,
                             with_scaled=with_scaled)
    if with_scaled:
        fn = body
    else:
        def fn(a1, a2, hh, dg, ct, w1, w2, w3, bb1, bb2, paa, c):
            body(a1, a2, hh, dg, ct, w1, w2, w3, bb1, bb2, paa, c, None)
    out = pl.pallas_call(
        fn,
        grid=(n // BLK,),
        in_specs=[row, row, row, col, col, wspec, wspec, wspec, bspec, bspec,
                  pl.BlockSpec((1, 1), lambda i: (0, 0))],
        out_specs=[row] * nout,
        out_shape=[jax.ShapeDtypeStruct((n, HIDDEN), jnp.float32)] * nout,
    )(agg1, agg2, h, deg, cnt, W1, W2, W3,
      b1.reshape(1, HIDDEN), b2.reshape(1, HIDDEN), pa.reshape(1, 1))
    return out if with_scaled else (out[0], None)


def kernel(x_checkin, x_poi, ei_seq, ei_visits, ei_visited, ei_spatial,
           Wpc, bpc, Wpp, bpp, prelu_a,
           l1_seq_W, l1_seq_b, l1_vis_Wl, l1_vis_bl, l1_vis_Wr,
           l1_vtd_Wl, l1_vtd_bl, l1_vtd_Wr, l1_sp_W, l1_sp_b,
           l2_seq_W, l2_seq_b, l2_vis_Wl, l2_vis_bl, l2_vis_Wr,
           l2_vtd_Wl, l2_vtd_bl, l2_vtd_Wr, l2_sp_W, l2_sp_b):
    pa = jnp.asarray(prelu_a, jnp.float32)
    s_seq, d_seq = _pad_edges(ei_seq[0], ei_seq[1], N_CHECKIN, E_PAD_BIG)
    s_vtd, d_vtd = _pad_edges(ei_visited[0], ei_visited[1], N_CHECKIN, E_PAD_BIG)
    s_vis, d_vis = _pad_edges(ei_visits[0], ei_visits[1], N_POI, E_PAD_BIG)
    s_sp, d_sp = _pad_edges(ei_spatial[0], ei_spatial[1], N_POI, E_PAD_SP)

    deg_seq, cnt_vtd, cnt_vis, deg_sp = _counts_k(d_seq, d_vtd, d_vis, d_sp)
    deg_seq = deg_seq.reshape(N_CHECKIN, 1)
    cnt_vtd = cnt_vtd.reshape(N_CHECKIN, 1)
    cnt_vis = cnt_vis.reshape(N_POI, 1)
    deg_sp = deg_sp.reshape(N_POI, 1)

    hc, hc_s = _transform(x_checkin, Wpc, bpc, deg_seq, N_CHECKIN)
    hp, hp_s = _transform(x_poi, Wpp, bpp, deg_sp, N_POI)

    agg_seq = _agg_cc(hc_s, s_seq, d_seq)
    agg_vtd = _agg_pc(hp, s_vtd, d_vtd)
    agg_vis = _agg_cp(hc, s_vis, d_vis)
    agg_sp = _agg_pp(hp_s, s_sp, d_sp)

    c1, c1_s = _combine(agg_seq, agg_vtd, hc, deg_seq, cnt_vtd,
                        l1_seq_W, l1_vtd_Wl, l1_vtd_Wr, l1_seq_b, l1_vtd_bl,
                        pa, N_CHECKIN, True, True)
    p1, p1_s = _combine(agg_sp, agg_vis, hp, deg_sp, cnt_vis,
                        l1_sp_W, l1_vis_Wl, l1_vis_Wr, l1_sp_b, l1_vis_bl,
                        pa, N_POI, True, True)

    agg_seq2 = _agg_cc(c1_s, s_seq, d_seq)
    agg_vtd2 = _agg_pc(p1, s_vtd, d_vtd)
    agg_vis2 = _agg_cp(c1, s_vis, d_vis)
    agg_sp2 = _agg_pp(p1_s, s_sp, d_sp)

    c2, _ = _combine(agg_seq2, agg_vtd2, c1, deg_seq, cnt_vtd,
                     l2_seq_W, l2_vtd_Wl, l2_vtd_Wr, l2_seq_b, l2_vtd_bl,
                     pa, N_CHECKIN, False, False)
    p2, _ = _combine(agg_sp2, agg_vis2, p1, deg_sp, cnt_vis,
                     l2_sp_W, l2_vis_Wl, l2_vis_Wr, l2_sp_b, l2_vis_bl,
                     pa, N_POI, False, False)
    return (c2, p2)


# speedup vs baseline: 1.5179x; 1.5179x over previous
import functools

import jax
import jax.numpy as jnp
from jax import lax
from jax.experimental import pallas as pl
from jax.experimental.pallas import tpu as pltpu
from jax.experimental.pallas import tpu_sc as plsc

N_CHECKIN = 100000
N_POI = 20000
HIDDEN = 128
BLK = 1000

C_EDGE = 1920
G = 128
NSUB = 16
BROWS = 10000
BPAD = 10240
E_PAD_BIG = 614400
E_PAD_SP = 337920


def _as_i32(x_bf16):
    n = x_bf16.shape[0]
    return lax.bitcast_convert_type(
        x_bf16.reshape(n, HIDDEN // 2, 2), jnp.int32)


def _pad_edges(src, dst, n_dst, e_pad):
    e = src.shape[0]
    pad = e_pad - e
    srcp = jnp.concatenate([src, jnp.zeros((pad,), jnp.int32)])
    dstp = jnp.concatenate([dst, jnp.full((pad,), n_dst, jnp.int32)])
    return srcp, dstp


_PERM = []
for _k in range(4):
    _PERM.extend(32 * _k + 2 * _j for _j in range(16))
    _PERM.extend(32 * _k + 2 * _j + 1 for _j in range(16))
_PERM = tuple(_PERM)


def _make_agg(n_src, n_dst, e_pad):
    nblk = n_dst // (2 * BROWS)
    et = e_pad // NSUB
    nch = et // C_EDGE
    nsc = C_EDGE // G
    mesh = plsc.VectorSubcoreMesh(core_axis_name="c", subcore_axis_name="s")

    @functools.partial(
        pl.kernel,
        out_type=jax.ShapeDtypeStruct((n_dst, HIDDEN), jnp.float32),
        mesh=mesh,
        compiler_params=pltpu.CompilerParams(needs_layout_passes=False,
                                             use_tc_tiling_on_sc=False),
        scratch_types=[
            pltpu.VMEM((C_EDGE,), jnp.int32),
            pltpu.VMEM((C_EDGE,), jnp.int32),
            pltpu.VMEM((C_EDGE + 16,), jnp.int32),
            pltpu.VMEM((C_EDGE + 16,), jnp.int32),
            pltpu.VMEM((nsc, G), jnp.int32),
            pltpu.VMEM((G, HIDDEN // 2), jnp.int32),
            pltpu.VMEM((G, HIDDEN // 2), jnp.int32),
            pltpu.VMEM((G, HIDDEN), jnp.float32),
            pltpu.VMEM_SHARED((BPAD, HIDDEN), jnp.float32),
            pltpu.SemaphoreType.DMA,
            pltpu.SemaphoreType.DMA,
        ],
    )
    def agg(table, srcp, dstp, out,
            dch, sch, s1d, d1d, d2d, rows0, rows1, rowsf, block, gsem0, gsem1):
        cid = lax.axis_index("c")
        sid = lax.axis_index("s")
        ebase = sid * et
        zero16f = jnp.zeros((16,), jnp.float32)
        zero16i = jnp.zeros((16,), jnp.int32)
        dum16 = jnp.full((16,), BROWS, jnp.int32)

        def zs_body(i, _):
            s1d[pl.ds(i * 16, 16)] = zero16i
            return 0
        lax.fori_loop(0, C_EDGE // 16, zs_body, 0)

        for blk in range(nblk):
            r0 = (cid * nblk + blk) * BROWS

            def zb_body(i, _):
                for k in range(HIDDEN // 16):
                    rowsf[i, pl.ds(k * 16, 16)] = zero16f
                return 0
            lax.fori_loop(0, G, zb_body, 0)
            for k in range(BPAD // NSUB // G):
                pltpu.sync_copy(rowsf, block.at[pl.ds(sid * (BPAD // NSUB)
                                                      + k * G, G)])
            plsc.subcore_barrier()

            def chunk_body(ch, _):
                base = ebase + ch * C_EDGE
                pltpu.sync_copy(dstp.at[pl.ds(base, C_EDGE)], dch)
                pltpu.sync_copy(srcp.at[pl.ds(base, C_EDGE)], sch)

                def zd_body(i, _):
                    d1d[pl.ds(i * 16, 16)] = dum16
                    return 0
                lax.fori_loop(0, C_EDGE // 16, zd_body, 0)

                def compact(i, cnt):
                    vd = dch[pl.ds(i * 16, 16)]
                    vs = sch[pl.ds(i * 16, 16)]
                    m = jnp.logical_and(vd >= r0, vd < r0 + BROWS)
                    cum = plsc.cumsum(m.astype(jnp.int32))
                    pos = jnp.where(m, cum - 1 + cnt, C_EDGE)
                    plsc.store_scatter(s1d, [pos], vs)
                    plsc.store_scatter(d1d, [pos], vd - r0)
                    return cnt + plsc.all_reduce_population_count(m)[0]
                cnt = lax.fori_loop(0, C_EDGE // 16, compact, jnp.int32(0))

                bufs = ((rows0, gsem0), (rows1, gsem1))
                descs = {}

                def fire_gather(j):
                    buf, sem = bufs[j % 2]
                    for k in range(G // 16):
                        d2d[j, pl.ds(k * 16, 16)] = \
                            d1d[pl.ds(j * G + k * 16, 16)]
                    descs[j] = pltpu.async_copy(
                        table.at[s1d.at[pl.ds(j * G, G)]], buf, sem)

                @pl.when(cnt > 0)
                def _():
                    fire_gather(0)

                for j in range(nsc):
                    @pl.when(j * G < cnt)
                    def _(j=j):
                        if j + 1 < nsc:
                            @pl.when((j + 1) * G < cnt)
                            def _():
                                fire_gather(j + 1)
                        descs[j].wait()
                        buf = bufs[j % 2][0]

                        def conv_body(i, _):
                            for k in range(HIDDEN // 32):
                                bfv = plsc.bitcast(
                                    buf[i, pl.ds(k * 16, 16)], jnp.bfloat16)
                                a, b = plsc.unpack(
                                    bfv, format=plsc.PackFormat.INTERLEAVED)
                                rowsf[i, pl.ds(32 * k, 16)] = a
                                rowsf[i, pl.ds(32 * k + 16, 16)] = b
                            return 0
                        lax.fori_loop(0, G, conv_body, 0)
                        pltpu.sync_copy(rowsf,
                                        block.at[d2d.at[j]], add=True)
                return 0
            lax.fori_loop(0, nch, chunk_body, 0)
            plsc.subcore_barrier()

            w = 80
            nchunks_wb = BROWS // w
            for k in range((nchunks_wb + NSUB - 1) // NSUB):
                idx = sid + k * NSUB

                @pl.when(idx < nchunks_wb)
                def _():
                    off = pl.multiple_of(idx * w, w)
                    pltpu.sync_copy(block.at[pl.ds(off, w)],
                                    rowsf.at[pl.ds(0, w)])
                    pltpu.sync_copy(rowsf.at[pl.ds(0, w)],
                                    out.at[pl.ds(r0 + off, w)])
            plsc.subcore_barrier()

    return agg


_EDGE_DEFS = (
    ("seq", N_CHECKIN, E_PAD_BIG),
    ("vtd", N_CHECKIN, E_PAD_BIG),
    ("vis", N_POI, E_PAD_BIG),
    ("sp", N_POI, E_PAD_SP),
)


def _make_counts():
    mesh = plsc.VectorSubcoreMesh(core_axis_name="c", subcore_axis_name="s")
    allocs = {N_CHECKIN: 50176, N_POI: 10240}

    @functools.partial(
        pl.kernel,
        out_type=[jax.ShapeDtypeStruct((n, ), jnp.float32)
                  for _, n, _ in _EDGE_DEFS],
        mesh=mesh,
        compiler_params=pltpu.CompilerParams(needs_layout_passes=False),
        scratch_types=[
            pltpu.VMEM((C_EDGE,), jnp.int32),
            pltpu.VMEM((C_EDGE + 16,), jnp.int32),
            pltpu.VMEM((C_EDGE // G, G), jnp.int32),
            pltpu.VMEM((G,), jnp.float32),
            pltpu.VMEM((50176 // NSUB,), jnp.float32),
            pltpu.VMEM((1000,), jnp.float32),
            pltpu.VMEM_SHARED((50176,), jnp.float32),
            pltpu.VMEM_SHARED((50176,), jnp.float32),
            pltpu.VMEM_SHARED((10240,), jnp.float32),
            pltpu.VMEM_SHARED((10240,), jnp.float32),
        ],
    )
    def counts(d_seq, d_vtd, d_vis, d_sp,
               o_seq, o_vtd, o_vis, o_sp,
               dch, d1d, d2d, ones, zbuf, wbuf, c0, c1, c2, c3):
        cid = lax.axis_index("c")
        sid = lax.axis_index("s")
        one16 = jnp.ones((16,), jnp.float32)
        zero16f = jnp.zeros((16,), jnp.float32)

        def zo_body(i, _):
            ones[pl.ds(i * 16, 16)] = one16
            return 0
        lax.fori_loop(0, G // 16, zo_body, 0)

        def zz_body(i, _):
            zbuf[pl.ds(i * 16, 16)] = zero16f
            return 0
        lax.fori_loop(0, 50176 // NSUB // 16, zz_body, 0)

        for (nm, n_dst, e_pad), dst_in, out_ref, cspm in zip(
                _EDGE_DEFS, (d_seq, d_vtd, d_vis, d_sp),
                (o_seq, o_vtd, o_vis, o_sp), (c0, c1, c2, c3)):
            nhalf = n_dst // 2
            alloc = allocs[n_dst]
            share = alloc // NSUB
            et = e_pad // NSUB
            nch = et // C_EDGE
            lo = cid * nhalf
            dum16 = jnp.full((16,), nhalf, jnp.int32)

            pltpu.sync_copy(zbuf.at[pl.ds(0, share)],
                            cspm.at[pl.ds(sid * share, share)])
            plsc.subcore_barrier()

            def chunk_body(ch, _):
                base = sid * et + ch * C_EDGE
                pltpu.sync_copy(dst_in.at[pl.ds(base, C_EDGE)], dch)

                def zd_body(i, _):
                    d1d[pl.ds(i * 16, 16)] = dum16
                    return 0
                lax.fori_loop(0, C_EDGE // 16, zd_body, 0)

                def compact(i, cnt):
                    vd = dch[pl.ds(i * 16, 16)] - lo
                    m = jnp.logical_and(vd >= 0, vd < nhalf)
                    cum = plsc.cumsum(m.astype(jnp.int32))
                    pos = jnp.where(m, cum - 1 + cnt, C_EDGE)
                    plsc.store_scatter(d1d, [pos], vd)
                    return cnt + plsc.all_reduce_population_count(m)[0]
                cnt = lax.fori_loop(0, C_EDGE // 16, compact, jnp.int32(0))

                def fire(j, _):
                    @pl.when(j * G < cnt)
                    def _():
                        for k in range(G // 16):
                            d2d[j, pl.ds(k * 16, 16)] = \
                                d1d[pl.ds(j * G + k * 16, 16)]
                        pltpu.sync_copy(ones, cspm.at[d2d.at[j]], add=True)
                    return 0
                lax.fori_loop(0, C_EDGE // G, fire, 0)
                return 0
            lax.fori_loop(0, nch, chunk_body, 0)
            plsc.subcore_barrier()

            nwb = nhalf // 1000
            for k in range((nwb + NSUB - 1) // NSUB):
                idx = sid + k * NSUB

                @pl.when(idx < nwb)
                def _():
                    pltpu.sync_copy(cspm.at[pl.ds(idx * 1000, 1000)], wbuf)
                    pltpu.sync_copy(wbuf, out_ref.at[pl.ds(lo + idx * 1000,
                                                           1000)])
            plsc.subcore_barrier()

    return counts


_agg_cc = _make_agg(N_CHECKIN, N_CHECKIN, E_PAD_BIG)
_agg_pc = _make_agg(N_POI, N_CHECKIN, E_PAD_BIG)
_agg_cp = _make_agg(N_CHECKIN, N_POI, E_PAD_BIG)
_agg_pp = _make_agg(N_POI, N_POI, E_PAD_SP)
_counts_k = _make_counts()


def _transform_body(x_ref, W_ref, b_ref, deg_ref, h_ref, hbf_ref, hsbf_ref):
    h = jnp.dot(x_ref[:], W_ref[:], preferred_element_type=jnp.float32,
                precision=lax.Precision.HIGHEST) + b_ref[:]
    deg = deg_ref[:]
    dis = jnp.where(deg > 0.0, lax.rsqrt(jnp.maximum(deg, 1e-12)), 0.0)
    h_ref[:] = h
    hbf_ref[:] = h.astype(jnp.bfloat16)
    hsbf_ref[:] = (dis * h).astype(jnp.bfloat16)


def _transform(x, W, b, deg, n):
    row = pl.BlockSpec((BLK, HIDDEN), lambda i: (i, 0))
    return pl.pallas_call(
        _transform_body,
        grid=(n // BLK,),
        in_specs=[
            row,
            pl.BlockSpec((HIDDEN, HIDDEN), lambda i: (0, 0)),
            pl.BlockSpec((1, HIDDEN), lambda i: (0, 0)),
            pl.BlockSpec((BLK, 1), lambda i: (i, 0)),
        ],
        out_specs=[row, row, row],
        out_shape=[jax.ShapeDtypeStruct((n, HIDDEN), jnp.float32),
                   jax.ShapeDtypeStruct((n, HIDDEN), jnp.bfloat16),
                   jax.ShapeDtypeStruct((n, HIDDEN), jnp.bfloat16)],
    )(x, W, b.reshape(1, HIDDEN), deg)


def _combine_body(agg1_ref, agg2_ref, h_ref, deg_ref, cnt_ref,
                  W1_ref, W2_ref, W3_ref, b1_ref, b2_ref, pa_ref,
                  c_ref, csc_ref, *, with_prelu, with_scaled):
    deg = deg_ref[:]
    dis = jnp.where(deg > 0.0, lax.rsqrt(jnp.maximum(deg, 1e-12)), 0.0)
    invc = 1.0 / jnp.maximum(cnt_ref[:], 1.0)
    hi = lax.Precision.HIGHEST
    t = dis * jnp.dot(agg1_ref[:], W1_ref[:],
                      preferred_element_type=jnp.float32, precision=hi)
    t = t + b1_ref[:] + b2_ref[:]
    t = t + jnp.dot(invc * agg2_ref[:], W2_ref[:],
                    preferred_element_type=jnp.float32, precision=hi)
    t = t + jnp.dot(h_ref[:], W3_ref[:],
                    preferred_element_type=jnp.float32, precision=hi)
    if with_prelu:
        t = jnp.where(t >= 0.0, t, pa_ref[0, 0] * t)
    c_ref[:] = t
    if with_scaled:
        cbf_ref, csbf_ref = csc_ref
        cbf_ref[:] = t.astype(jnp.bfloat16)
        csbf_ref[:] = (dis * t).astype(jnp.bfloat16)


def _combine(agg1, agg2, h, deg, cnt, W1, W2, W3, b1, b2, pa, n,
             with_prelu, with_scaled):
    row = pl.BlockSpec((BLK, HIDDEN), lambda i: (i, 0))
    wspec = pl.BlockSpec((HIDDEN, HIDDEN), lambda i: (0, 0))
    bspec = pl.BlockSpec((1, HIDDEN), lambda i: (0, 0))
    col = pl.BlockSpec((BLK, 1), lambda i: (i, 0))
    body = functools.partial(_combine_body, with_prelu=with_prelu,
                             with_scaled=with_scaled)
    if with_scaled:
        def fn(a1, a2, hh, dg, ct, w1, w2, w3, bb1, bb2, paa, c, cbf, csbf):
            body(a1, a2, hh, dg, ct, w1, w2, w3, bb1, bb2, paa, c,
                 (cbf, csbf))
        nout = 3
        out_shape = [jax.ShapeDtypeStruct((n, HIDDEN), jnp.float32),
                     jax.ShapeDtypeStruct((n, HIDDEN), jnp.bfloat16),
                     jax.ShapeDtypeStruct((n, HIDDEN), jnp.bfloat16)]
    else:
        def fn(a1, a2, hh, dg, ct, w1, w2, w3, bb1, bb2, paa, c):
            body(a1, a2, hh, dg, ct, w1, w2, w3, bb1, bb2, paa, c, None)
        nout = 1
        out_shape = [jax.ShapeDtypeStruct((n, HIDDEN), jnp.float32)]
    out = pl.pallas_call(
        fn,
        grid=(n // BLK,),
        in_specs=[row, row, row, col, col, wspec, wspec, wspec, bspec, bspec,
                  pl.BlockSpec((1, 1), lambda i: (0, 0))],
        out_specs=[row] * nout,
        out_shape=out_shape,
    )(agg1, agg2, h, deg, cnt, W1, W2, W3,
      b1.reshape(1, HIDDEN), b2.reshape(1, HIDDEN), pa.reshape(1, 1))
    return out if with_scaled else (out[0], None, None)


def kernel(x_checkin, x_poi, ei_seq, ei_visits, ei_visited, ei_spatial,
           Wpc, bpc, Wpp, bpp, prelu_a,
           l1_seq_W, l1_seq_b, l1_vis_Wl, l1_vis_bl, l1_vis_Wr,
           l1_vtd_Wl, l1_vtd_bl, l1_vtd_Wr, l1_sp_W, l1_sp_b,
           l2_seq_W, l2_seq_b, l2_vis_Wl, l2_vis_bl, l2_vis_Wr,
           l2_vtd_Wl, l2_vtd_bl, l2_vtd_Wr, l2_sp_W, l2_sp_b):
    pa = jnp.asarray(prelu_a, jnp.float32)
    s_seq, d_seq = _pad_edges(ei_seq[0], ei_seq[1], N_CHECKIN, E_PAD_BIG)
    s_vtd, d_vtd = _pad_edges(ei_visited[0], ei_visited[1], N_CHECKIN, E_PAD_BIG)
    s_vis, d_vis = _pad_edges(ei_visits[0], ei_visits[1], N_POI, E_PAD_BIG)
    s_sp, d_sp = _pad_edges(ei_spatial[0], ei_spatial[1], N_POI, E_PAD_SP)

    deg_seq, cnt_vtd, cnt_vis, deg_sp = _counts_k(d_seq, d_vtd, d_vis, d_sp)
    deg_seq = deg_seq.reshape(N_CHECKIN, 1)
    cnt_vtd = cnt_vtd.reshape(N_CHECKIN, 1)
    cnt_vis = cnt_vis.reshape(N_POI, 1)
    deg_sp = deg_sp.reshape(N_POI, 1)

    perm = jnp.asarray(_PERM, jnp.int32)

    hc, hc_bf, hcs_bf = _transform(x_checkin, Wpc, bpc, deg_seq, N_CHECKIN)
    hp, hp_bf, hps_bf = _transform(x_poi, Wpp, bpp, deg_sp, N_POI)

    agg_seq = _agg_cc(_as_i32(hcs_bf), s_seq, d_seq)
    agg_vtd = _agg_pc(_as_i32(hp_bf), s_vtd, d_vtd)
    agg_vis = _agg_cp(_as_i32(hc_bf), s_vis, d_vis)
    agg_sp = _agg_pp(_as_i32(hps_bf), s_sp, d_sp)

    c1, c1_bf, c1s_bf = _combine(
        agg_seq, agg_vtd, hc, deg_seq, cnt_vtd,
        l1_seq_W[perm], l1_vtd_Wl[perm], l1_vtd_Wr, l1_seq_b, l1_vtd_bl,
        pa, N_CHECKIN, True, True)
    p1, p1_bf, p1s_bf = _combine(
        agg_sp, agg_vis, hp, deg_sp, cnt_vis,
        l1_sp_W[perm], l1_vis_Wl[perm], l1_vis_Wr, l1_sp_b, l1_vis_bl,
        pa, N_POI, True, True)

    agg_seq2 = _agg_cc(_as_i32(c1s_bf), s_seq, d_seq)
    agg_vtd2 = _agg_pc(_as_i32(p1_bf), s_vtd, d_vtd)
    agg_vis2 = _agg_cp(_as_i32(c1_bf), s_vis, d_vis)
    agg_sp2 = _agg_pp(_as_i32(p1s_bf), s_sp, d_sp)

    c2, _, _ = _combine(
        agg_seq2, agg_vtd2, c1, deg_seq, cnt_vtd,
        l2_seq_W[perm], l2_vtd_Wl[perm], l2_vtd_Wr, l2_seq_b, l2_vtd_bl,
        pa, N_CHECKIN, False, False)
    p2, _, _ = _combine(
        agg_sp2, agg_vis2, p1, deg_sp, cnt_vis,
        l2_sp_W[perm], l2_vis_Wl[perm], l2_vis_Wr, l2_sp_b, l2_vis_bl,
        pa, N_POI, False, False)
    return (c2, p2)


# speedup vs baseline: 2.2768x; 1.5000x over previous
import functools

import jax
import jax.numpy as jnp
from jax import lax
from jax.experimental import pallas as pl
from jax.experimental.pallas import tpu as pltpu
from jax.experimental.pallas import tpu_sc as plsc

N_CHECKIN = 100000
N_POI = 20000
HIDDEN = 128
BLK = 1000

C_EDGE = 1920
G = 64
NSUB = 16
BROWS = 10000
BPAD = 10240
E_PAD_BIG = 614400
E_PAD_SP = 337920


def _as_i32(x_bf16):
    n = x_bf16.shape[0]
    return lax.bitcast_convert_type(
        x_bf16.reshape(n, HIDDEN // 2, 2), jnp.int32)


def _pad_edges(src, dst, n_dst, e_pad):
    e = src.shape[0]
    pad = e_pad - e
    srcp = jnp.concatenate([src, jnp.zeros((pad,), jnp.int32)])
    dstp = jnp.concatenate([dst, jnp.full((pad,), n_dst, jnp.int32)])
    return srcp, dstp


_PERM = []
for _k in range(4):
    _PERM.extend(32 * _k + 2 * _j for _j in range(16))
    _PERM.extend(32 * _k + 2 * _j + 1 for _j in range(16))
_PERM = tuple(_PERM)


def _make_agg(n_src, n_dst, e_pad):
    nblk = n_dst // (2 * BROWS)
    et = e_pad // NSUB
    nch = et // C_EDGE
    nsc = C_EDGE // G
    mesh = plsc.VectorSubcoreMesh(core_axis_name="c", subcore_axis_name="s")

    @functools.partial(
        pl.kernel,
        out_type=jax.ShapeDtypeStruct((n_dst, HIDDEN), jnp.float32),
        mesh=mesh,
        compiler_params=pltpu.CompilerParams(needs_layout_passes=False,
                                             use_tc_tiling_on_sc=False),
        scratch_types=[
            pltpu.VMEM((C_EDGE,), jnp.int32),
            pltpu.VMEM((C_EDGE,), jnp.int32),
            pltpu.VMEM((C_EDGE + 16,), jnp.int32),
            pltpu.VMEM((C_EDGE + 16,), jnp.int32),
            pltpu.VMEM((nsc, G), jnp.int32),
            pltpu.VMEM((G, HIDDEN // 2), jnp.int32),
            pltpu.VMEM((G, HIDDEN // 2), jnp.int32),
            pltpu.VMEM((G, HIDDEN), jnp.float32),
            pltpu.VMEM((G, HIDDEN), jnp.float32),
            pltpu.VMEM_SHARED((BPAD, HIDDEN), jnp.float32),
            pltpu.SemaphoreType.DMA,
            pltpu.SemaphoreType.DMA,
            pltpu.SemaphoreType.DMA,
            pltpu.SemaphoreType.DMA,
        ],
    )
    def agg(table, srcp, dstp, out,
            dch, sch, s1d, d1d, d2d, rows0, rows1, rowsf0, rowsf1, block,
            gsem0, gsem1, ssem0, ssem1):
        cid = lax.axis_index("c")
        sid = lax.axis_index("s")
        ebase = sid * et
        zero16f = jnp.zeros((16,), jnp.float32)
        zero16i = jnp.zeros((16,), jnp.int32)
        dum16 = jnp.full((16,), BROWS, jnp.int32)

        def zs_body(i, _):
            s1d[pl.ds(i * 16, 16)] = zero16i
            return 0
        lax.fori_loop(0, C_EDGE // 16, zs_body, 0)

        def blk_body(blk, _):
            r0 = (cid * nblk + blk) * BROWS

            def zb_body(i, _):
                for k in range(HIDDEN // 16):
                    rowsf0[i, pl.ds(k * 16, 16)] = zero16f
                return 0
            lax.fori_loop(0, G, zb_body, 0)
            for k in range(BPAD // NSUB // G):
                pltpu.sync_copy(rowsf0, block.at[pl.ds(sid * (BPAD // NSUB)
                                                       + k * G, G)])
            plsc.subcore_barrier()

            def chunk_body(ch, _):
                base = ebase + ch * C_EDGE
                pltpu.sync_copy(dstp.at[pl.ds(base, C_EDGE)], dch)
                pltpu.sync_copy(srcp.at[pl.ds(base, C_EDGE)], sch)

                def zd_body(i, _):
                    d1d[pl.ds(i * 16, 16)] = dum16
                    return 0
                lax.fori_loop(0, C_EDGE // 16, zd_body, 0)

                def compact(i, cnt):
                    vd = dch[pl.ds(i * 16, 16)]
                    vs = sch[pl.ds(i * 16, 16)]
                    m = jnp.logical_and(vd >= r0, vd < r0 + BROWS)
                    cum = plsc.cumsum(m.astype(jnp.int32))
                    pos = jnp.where(m, cum - 1 + cnt, C_EDGE)
                    plsc.store_scatter(s1d, [pos], vs)
                    plsc.store_scatter(d1d, [pos], vd - r0)
                    return cnt + plsc.all_reduce_population_count(m)[0]
                cnt = lax.fori_loop(0, C_EDGE // 16, compact, jnp.int32(0))

                bufs_g = ((rows0, gsem0), (rows1, gsem1))
                bufs_f = ((rowsf0, ssem0), (rowsf1, ssem1))
                descs_g = {}
                descs_s = {}

                def fire_gather(j):
                    buf, sem = bufs_g[j % 2]
                    for k in range(G // 16):
                        d2d[j, pl.ds(k * 16, 16)] = \
                            d1d[pl.ds(j * G + k * 16, 16)]
                    descs_g[j] = pltpu.async_copy(
                        table.at[s1d.at[pl.ds(j * G, G)]], buf, sem)

                @pl.when(cnt > 0)
                def _():
                    fire_gather(0)

                for j in range(nsc + 2):
                    if j >= 2:
                        @pl.when((j - 2) * G < cnt)
                        def _(j=j):
                            descs_s[j - 2].wait()
                    if j < nsc:
                        @pl.when(j * G < cnt)
                        def _(j=j):
                            if j + 1 < nsc:
                                @pl.when((j + 1) * G < cnt)
                                def _():
                                    fire_gather(j + 1)
                            descs_g[j].wait()
                            gbuf = bufs_g[j % 2][0]
                            fbuf, fsem = bufs_f[j % 2]

                            def conv_body(i, _):
                                for k in range(HIDDEN // 32):
                                    bfv = plsc.bitcast(
                                        gbuf[i, pl.ds(k * 16, 16)],
                                        jnp.bfloat16)
                                    a, b = plsc.unpack(
                                        bfv,
                                        format=plsc.PackFormat.INTERLEAVED)
                                    fbuf[i, pl.ds(32 * k, 16)] = a
                                    fbuf[i, pl.ds(32 * k + 16, 16)] = b
                                return 0
                            lax.fori_loop(0, G, conv_body, 0)
                            descs_s[j] = pltpu.async_copy(
                                fbuf, block.at[d2d.at[j]], fsem, add=True)
                return 0
            lax.fori_loop(0, nch, chunk_body, 0)
            plsc.subcore_barrier()

            w = 40
            nchunks_wb = BROWS // w
            for k in range((nchunks_wb + NSUB - 1) // NSUB):
                idx = sid + k * NSUB

                @pl.when(idx < nchunks_wb)
                def _():
                    off = pl.multiple_of(idx * w, w)
                    pltpu.sync_copy(block.at[pl.ds(off, w)],
                                    rowsf0.at[pl.ds(0, w)])
                    pltpu.sync_copy(rowsf0.at[pl.ds(0, w)],
                                    out.at[pl.ds(r0 + off, w)])
            plsc.subcore_barrier()
            return 0
        lax.fori_loop(0, nblk, blk_body, 0)

    return agg


_EDGE_DEFS = (
    ("seq", N_CHECKIN, E_PAD_BIG),
    ("vtd", N_CHECKIN, E_PAD_BIG),
    ("vis", N_POI, E_PAD_BIG),
    ("sp", N_POI, E_PAD_SP),
)


def _make_counts():
    mesh = plsc.VectorSubcoreMesh(core_axis_name="c", subcore_axis_name="s")
    allocs = {N_CHECKIN: 50176, N_POI: 10240}

    @functools.partial(
        pl.kernel,
        out_type=[jax.ShapeDtypeStruct((n, ), jnp.float32)
                  for _, n, _ in _EDGE_DEFS],
        mesh=mesh,
        compiler_params=pltpu.CompilerParams(needs_layout_passes=False),
        scratch_types=[
            pltpu.VMEM((C_EDGE,), jnp.int32),
            pltpu.VMEM((C_EDGE + 16,), jnp.int32),
            pltpu.VMEM((C_EDGE // G, G), jnp.int32),
            pltpu.VMEM((G,), jnp.float32),
            pltpu.VMEM((50176 // NSUB,), jnp.float32),
            pltpu.VMEM((1000,), jnp.float32),
            pltpu.VMEM_SHARED((50176,), jnp.float32),
            pltpu.VMEM_SHARED((50176,), jnp.float32),
            pltpu.VMEM_SHARED((10240,), jnp.float32),
            pltpu.VMEM_SHARED((10240,), jnp.float32),
        ],
    )
    def counts(d_seq, d_vtd, d_vis, d_sp,
               o_seq, o_vtd, o_vis, o_sp,
               dch, d1d, d2d, ones, zbuf, wbuf, c0, c1, c2, c3):
        cid = lax.axis_index("c")
        sid = lax.axis_index("s")
        one16 = jnp.ones((16,), jnp.float32)
        zero16f = jnp.zeros((16,), jnp.float32)

        def zo_body(i, _):
            ones[pl.ds(i * 16, 16)] = one16
            return 0
        lax.fori_loop(0, G // 16, zo_body, 0)

        def zz_body(i, _):
            zbuf[pl.ds(i * 16, 16)] = zero16f
            return 0
        lax.fori_loop(0, 50176 // NSUB // 16, zz_body, 0)

        for (nm, n_dst, e_pad), dst_in, out_ref, cspm in zip(
                _EDGE_DEFS, (d_seq, d_vtd, d_vis, d_sp),
                (o_seq, o_vtd, o_vis, o_sp), (c0, c1, c2, c3)):
            nhalf = n_dst // 2
            alloc = allocs[n_dst]
            share = alloc // NSUB
            et = e_pad // NSUB
            nch = et // C_EDGE
            lo = cid * nhalf
            dum16 = jnp.full((16,), nhalf, jnp.int32)

            pltpu.sync_copy(zbuf.at[pl.ds(0, share)],
                            cspm.at[pl.ds(sid * share, share)])
            plsc.subcore_barrier()

            def chunk_body(ch, _):
                base = sid * et + ch * C_EDGE
                pltpu.sync_copy(dst_in.at[pl.ds(base, C_EDGE)], dch)

                def zd_body(i, _):
                    d1d[pl.ds(i * 16, 16)] = dum16
                    return 0
                lax.fori_loop(0, C_EDGE // 16, zd_body, 0)

                def compact(i, cnt):
                    vd = dch[pl.ds(i * 16, 16)] - lo
                    m = jnp.logical_and(vd >= 0, vd < nhalf)
                    cum = plsc.cumsum(m.astype(jnp.int32))
                    pos = jnp.where(m, cum - 1 + cnt, C_EDGE)
                    plsc.store_scatter(d1d, [pos], vd)
                    return cnt + plsc.all_reduce_population_count(m)[0]
                cnt = lax.fori_loop(0, C_EDGE // 16, compact, jnp.int32(0))

                def fire(j, _):
                    @pl.when(j * G < cnt)
                    def _():
                        for k in range(G // 16):
                            d2d[j, pl.ds(k * 16, 16)] = \
                                d1d[pl.ds(j * G + k * 16, 16)]
                        pltpu.sync_copy(ones, cspm.at[d2d.at[j]], add=True)
                    return 0
                lax.fori_loop(0, C_EDGE // G, fire, 0)
                return 0
            lax.fori_loop(0, nch, chunk_body, 0)
            plsc.subcore_barrier()

            nwb = nhalf // 1000
            for k in range((nwb + NSUB - 1) // NSUB):
                idx = sid + k * NSUB

                @pl.when(idx < nwb)
                def _():
                    pltpu.sync_copy(cspm.at[pl.ds(idx * 1000, 1000)], wbuf)
                    pltpu.sync_copy(wbuf, out_ref.at[pl.ds(lo + idx * 1000,
                                                           1000)])
            plsc.subcore_barrier()

    return counts


_agg_cc = _make_agg(N_CHECKIN, N_CHECKIN, E_PAD_BIG)
_agg_pc = _make_agg(N_POI, N_CHECKIN, E_PAD_BIG)
_agg_cp = _make_agg(N_CHECKIN, N_POI, E_PAD_BIG)
_agg_pp = _make_agg(N_POI, N_POI, E_PAD_SP)
_counts_k = _make_counts()


def _transform_body(x_ref, W_ref, b_ref, deg_ref, h_ref, hbf_ref, hsbf_ref):
    h = jnp.dot(x_ref[:], W_ref[:], preferred_element_type=jnp.float32,
                precision=lax.Precision.HIGHEST) + b_ref[:]
    deg = deg_ref[:]
    dis = jnp.where(deg > 0.0, lax.rsqrt(jnp.maximum(deg, 1e-12)), 0.0)
    h_ref[:] = h
    hbf_ref[:] = h.astype(jnp.bfloat16)
    hsbf_ref[:] = (dis * h).astype(jnp.bfloat16)


def _transform(x, W, b, deg, n):
    row = pl.BlockSpec((BLK, HIDDEN), lambda i: (i, 0))
    return pl.pallas_call(
        _transform_body,
        grid=(n // BLK,),
        in_specs=[
            row,
            pl.BlockSpec((HIDDEN, HIDDEN), lambda i: (0, 0)),
            pl.BlockSpec((1, HIDDEN), lambda i: (0, 0)),
            pl.BlockSpec((BLK, 1), lambda i: (i, 0)),
        ],
        out_specs=[row, row, row],
        out_shape=[jax.ShapeDtypeStruct((n, HIDDEN), jnp.float32),
                   jax.ShapeDtypeStruct((n, HIDDEN), jnp.bfloat16),
                   jax.ShapeDtypeStruct((n, HIDDEN), jnp.bfloat16)],
    )(x, W, b.reshape(1, HIDDEN), deg)


def _combine_body(agg1_ref, agg2_ref, h_ref, deg_ref, cnt_ref,
                  W1_ref, W2_ref, W3_ref, b1_ref, b2_ref, pa_ref,
                  c_ref, csc_ref, *, with_prelu, with_scaled):
    deg = deg_ref[:]
    dis = jnp.where(deg > 0.0, lax.rsqrt(jnp.maximum(deg, 1e-12)), 0.0)
    invc = 1.0 / jnp.maximum(cnt_ref[:], 1.0)
    hi = lax.Precision.HIGHEST
    t = dis * jnp.dot(agg1_ref[:], W1_ref[:],
                      preferred_element_type=jnp.float32, precision=hi)
    t = t + b1_ref[:] + b2_ref[:]
    t = t + jnp.dot(invc * agg2_ref[:], W2_ref[:],
                    preferred_element_type=jnp.float32, precision=hi)
    t = t + jnp.dot(h_ref[:], W3_ref[:],
                    preferred_element_type=jnp.float32, precision=hi)
    if with_prelu:
        t = jnp.where(t >= 0.0, t, pa_ref[0, 0] * t)
    c_ref[:] = t
    if with_scaled:
        cbf_ref, csbf_ref = csc_ref
        cbf_ref[:] = t.astype(jnp.bfloat16)
        csbf_ref[:] = (dis * t).astype(jnp.bfloat16)


def _combine(agg1, agg2, h, deg, cnt, W1, W2, W3, b1, b2, pa, n,
             with_prelu, with_scaled):
    row = pl.BlockSpec((BLK, HIDDEN), lambda i: (i, 0))
    wspec = pl.BlockSpec((HIDDEN, HIDDEN), lambda i: (0, 0))
    bspec = pl.BlockSpec((1, HIDDEN), lambda i: (0, 0))
    col = pl.BlockSpec((BLK, 1), lambda i: (i, 0))
    body = functools.partial(_combine_body, with_prelu=with_prelu,
                             with_scaled=with_scaled)
    if with_scaled:
        def fn(a1, a2, hh, dg, ct, w1, w2, w3, bb1, bb2, paa, c, cbf, csbf):
            body(a1, a2, hh, dg, ct, w1, w2, w3, bb1, bb2, paa, c,
                 (cbf, csbf))
        nout = 3
        out_shape = [jax.ShapeDtypeStruct((n, HIDDEN), jnp.float32),
                     jax.ShapeDtypeStruct((n, HIDDEN), jnp.bfloat16),
                     jax.ShapeDtypeStruct((n, HIDDEN), jnp.bfloat16)]
    else:
        def fn(a1, a2, hh, dg, ct, w1, w2, w3, bb1, bb2, paa, c):
            body(a1, a2, hh, dg, ct, w1, w2, w3, bb1, bb2, paa, c, None)
        nout = 1
        out_shape = [jax.ShapeDtypeStruct((n, HIDDEN), jnp.float32)]
    out = pl.pallas_call(
        fn,
        grid=(n // BLK,),
        in_specs=[row, row, row, col, col, wspec, wspec, wspec, bspec, bspec,
                  pl.BlockSpec((1, 1), lambda i: (0, 0))],
        out_specs=[row] * nout,
        out_shape=out_shape,
    )(agg1, agg2, h, deg, cnt, W1, W2, W3,
      b1.reshape(1, HIDDEN), b2.reshape(1, HIDDEN), pa.reshape(1, 1))
    return out if with_scaled else (out[0], None, None)


def kernel(x_checkin, x_poi, ei_seq, ei_visits, ei_visited, ei_spatial,
           Wpc, bpc, Wpp, bpp, prelu_a,
           l1_seq_W, l1_seq_b, l1_vis_Wl, l1_vis_bl, l1_vis_Wr,
           l1_vtd_Wl, l1_vtd_bl, l1_vtd_Wr, l1_sp_W, l1_sp_b,
           l2_seq_W, l2_seq_b, l2_vis_Wl, l2_vis_bl, l2_vis_Wr,
           l2_vtd_Wl, l2_vtd_bl, l2_vtd_Wr, l2_sp_W, l2_sp_b):
    pa = jnp.asarray(prelu_a, jnp.float32)
    s_seq, d_seq = _pad_edges(ei_seq[0], ei_seq[1], N_CHECKIN, E_PAD_BIG)
    s_vtd, d_vtd = _pad_edges(ei_visited[0], ei_visited[1], N_CHECKIN, E_PAD_BIG)
    s_vis, d_vis = _pad_edges(ei_visits[0], ei_visits[1], N_POI, E_PAD_BIG)
    s_sp, d_sp = _pad_edges(ei_spatial[0], ei_spatial[1], N_POI, E_PAD_SP)

    deg_seq, cnt_vtd, cnt_vis, deg_sp = _counts_k(d_seq, d_vtd, d_vis, d_sp)
    deg_seq = deg_seq.reshape(N_CHECKIN, 1)
    cnt_vtd = cnt_vtd.reshape(N_CHECKIN, 1)
    cnt_vis = cnt_vis.reshape(N_POI, 1)
    deg_sp = deg_sp.reshape(N_POI, 1)

    perm = jnp.asarray(_PERM, jnp.int32)

    hc, hc_bf, hcs_bf = _transform(x_checkin, Wpc, bpc, deg_seq, N_CHECKIN)
    hp, hp_bf, hps_bf = _transform(x_poi, Wpp, bpp, deg_sp, N_POI)

    agg_seq = _agg_cc(_as_i32(hcs_bf), s_seq, d_seq)
    agg_vtd = _agg_pc(_as_i32(hp_bf), s_vtd, d_vtd)
    agg_vis = _agg_cp(_as_i32(hc_bf), s_vis, d_vis)
    agg_sp = _agg_pp(_as_i32(hps_bf), s_sp, d_sp)

    c1, c1_bf, c1s_bf = _combine(
        agg_seq, agg_vtd, hc, deg_seq, cnt_vtd,
        l1_seq_W[perm], l1_vtd_Wl[perm], l1_vtd_Wr, l1_seq_b, l1_vtd_bl,
        pa, N_CHECKIN, True, True)
    p1, p1_bf, p1s_bf = _combine(
        agg_sp, agg_vis, hp, deg_sp, cnt_vis,
        l1_sp_W[perm], l1_vis_Wl[perm], l1_vis_Wr, l1_sp_b, l1_vis_bl,
        pa, N_POI, True, True)

    agg_seq2 = _agg_cc(_as_i32(c1s_bf), s_seq, d_seq)
    agg_vtd2 = _agg_pc(_as_i32(p1_bf), s_vtd, d_vtd)
    agg_vis2 = _agg_cp(_as_i32(c1_bf), s_vis, d_vis)
    agg_sp2 = _agg_pp(_as_i32(p1s_bf), s_sp, d_sp)

    c2, _, _ = _combine(
        agg_seq2, agg_vtd2, c1, deg_seq, cnt_vtd,
        l2_seq_W[perm], l2_vtd_Wl[perm], l2_vtd_Wr, l2_seq_b, l2_vtd_bl,
        pa, N_CHECKIN, False, False)
    p2, _, _ = _combine(
        agg_sp2, agg_vis2, p1, deg_sp, cnt_vis,
        l2_sp_W[perm], l2_vis_Wl[perm], l2_vis_Wr, l2_sp_b, l2_vis_bl,
        pa, N_POI, False, False)
    return (c2, p2)


# speedup vs baseline: 2.3932x; 1.0511x over previous
import functools

import jax
import jax.numpy as jnp
from jax import lax
from jax.experimental import pallas as pl
from jax.experimental.pallas import tpu as pltpu
from jax.experimental.pallas import tpu_sc as plsc

N_CHECKIN = 100000
N_POI = 20000
HIDDEN = 128
BLK = 1000

C_EDGE = 1920
G = 64
NSUB = 16
BROWS = 10000
BPAD = 10240
E_PAD_BIG = 614400
E_PAD_SP = 337920


def _as_i32(x_bf16):
    n = x_bf16.shape[0]
    return lax.bitcast_convert_type(
        x_bf16.reshape(n, HIDDEN // 2, 2), jnp.int32)


def _pad_edges(src, dst, n_dst, e_pad):
    e = src.shape[0]
    pad = e_pad - e
    srcp = jnp.concatenate([src, jnp.zeros((pad,), jnp.int32)])
    dstp = jnp.concatenate([dst, jnp.full((pad,), n_dst, jnp.int32)])
    return srcp, dstp


_PERM = []
for _k in range(4):
    _PERM.extend(32 * _k + 2 * _j for _j in range(16))
    _PERM.extend(32 * _k + 2 * _j + 1 for _j in range(16))
_PERM = tuple(_PERM)


def _make_agg(n_src, n_dst, e_pad):
    nblk = n_dst // (2 * BROWS)
    et = e_pad // NSUB
    nch = et // C_EDGE
    nsc = C_EDGE // G
    mesh = plsc.VectorSubcoreMesh(core_axis_name="c", subcore_axis_name="s")

    @functools.partial(
        pl.kernel,
        out_type=jax.ShapeDtypeStruct((n_dst, HIDDEN), jnp.float32),
        mesh=mesh,
        compiler_params=pltpu.CompilerParams(needs_layout_passes=False,
                                             use_tc_tiling_on_sc=False),
        scratch_types=[
            pltpu.VMEM((C_EDGE,), jnp.int32),
            pltpu.VMEM((C_EDGE,), jnp.int32),
            pltpu.VMEM((C_EDGE + 16,), jnp.int32),
            pltpu.VMEM((C_EDGE + 16,), jnp.int32),
            pltpu.VMEM((nsc, G), jnp.int32),
            pltpu.VMEM((G, HIDDEN // 2), jnp.int32),
            pltpu.VMEM((G, HIDDEN // 2), jnp.int32),
            pltpu.VMEM((G, HIDDEN), jnp.bfloat16),
            pltpu.VMEM((G, HIDDEN), jnp.bfloat16),
            pltpu.VMEM((G, HIDDEN), jnp.float32),
            pltpu.VMEM_SHARED((BPAD, HIDDEN), jnp.bfloat16),
            pltpu.SemaphoreType.DMA,
            pltpu.SemaphoreType.DMA,
            pltpu.SemaphoreType.DMA,
            pltpu.SemaphoreType.DMA,
        ],
    )
    def agg(table, srcp, dstp, out,
            dch, sch, s1d, d1d, d2d, rows0, rows1, rowsf0, rowsf1, rowswb,
            block, gsem0, gsem1, ssem0, ssem1):
        cid = lax.axis_index("c")
        sid = lax.axis_index("s")
        ebase = sid * et
        zero16f = jnp.zeros((16,), jnp.float32)
        zero16i = jnp.zeros((16,), jnp.int32)
        dum16 = jnp.full((16,), BROWS, jnp.int32)

        def zs_body(i, _):
            s1d[pl.ds(i * 16, 16)] = zero16i
            return 0
        lax.fori_loop(0, C_EDGE // 16, zs_body, 0)

        def blk_body(blk, _):
            r0 = (cid * nblk + blk) * BROWS

            zero32b = jnp.zeros((32,), jnp.bfloat16)

            def zb_body(i, _):
                for k in range(HIDDEN // 32):
                    rowsf0[i, pl.ds(k * 32, 32)] = zero32b
                return 0
            lax.fori_loop(0, G, zb_body, 0)
            for k in range(BPAD // NSUB // G):
                pltpu.sync_copy(rowsf0, block.at[pl.ds(sid * (BPAD // NSUB)
                                                       + k * G, G)])
            plsc.subcore_barrier()

            def chunk_body(ch, _):
                base = ebase + ch * C_EDGE
                pltpu.sync_copy(dstp.at[pl.ds(base, C_EDGE)], dch)
                pltpu.sync_copy(srcp.at[pl.ds(base, C_EDGE)], sch)

                def zd_body(i, _):
                    d1d[pl.ds(i * 16, 16)] = dum16
                    return 0
                lax.fori_loop(0, C_EDGE // 16, zd_body, 0)

                def compact(i, cnt):
                    vd = dch[pl.ds(i * 16, 16)]
                    vs = sch[pl.ds(i * 16, 16)]
                    m = jnp.logical_and(vd >= r0, vd < r0 + BROWS)
                    cum = plsc.cumsum(m.astype(jnp.int32))
                    pos = jnp.where(m, cum - 1 + cnt, C_EDGE)
                    plsc.store_scatter(s1d, [pos], vs)
                    plsc.store_scatter(d1d, [pos], vd - r0)
                    return cnt + plsc.all_reduce_population_count(m)[0]
                cnt = lax.fori_loop(0, C_EDGE // 16, compact, jnp.int32(0))

                bufs_g = ((rows0, gsem0), (rows1, gsem1))
                bufs_f = ((rowsf0, ssem0), (rowsf1, ssem1))
                descs_g = {}
                descs_s = {}

                def fire_gather(j):
                    buf, sem = bufs_g[j % 2]
                    for k in range(G // 16):
                        d2d[j, pl.ds(k * 16, 16)] = \
                            d1d[pl.ds(j * G + k * 16, 16)]
                    descs_g[j] = pltpu.async_copy(
                        table.at[s1d.at[pl.ds(j * G, G)]], buf, sem)

                @pl.when(cnt > 0)
                def _():
                    fire_gather(0)

                for j in range(nsc + 2):
                    if j >= 2:
                        @pl.when((j - 2) * G < cnt)
                        def _(j=j):
                            descs_s[j - 2].wait()
                    if j < nsc:
                        @pl.when(j * G < cnt)
                        def _(j=j):
                            if j + 1 < nsc:
                                @pl.when((j + 1) * G < cnt)
                                def _():
                                    fire_gather(j + 1)
                            descs_g[j].wait()
                            gbuf = bufs_g[j % 2][0]
                            fbuf, fsem = bufs_f[j % 2]

                            def conv_body(i, _):
                                for k in range(HIDDEN // 32):
                                    fbuf[i, pl.ds(32 * k, 32)] = plsc.bitcast(
                                        gbuf[i, pl.ds(k * 16, 16)],
                                        jnp.bfloat16)
                                return 0
                            lax.fori_loop(0, G, conv_body, 0)
                            descs_s[j] = pltpu.async_copy(
                                fbuf, block.at[d2d.at[j]], fsem, add=True)
                return 0
            lax.fori_loop(0, nch, chunk_body, 0)
            plsc.subcore_barrier()

            w = 40
            nchunks_wb = BROWS // w
            for k in range((nchunks_wb + NSUB - 1) // NSUB):
                idx = sid + k * NSUB

                @pl.when(idx < nchunks_wb)
                def _():
                    off = pl.multiple_of(idx * w, w)
                    pltpu.sync_copy(block.at[pl.ds(off, w)],
                                    rowsf0.at[pl.ds(0, w)])

                    def wb_body(i, _):
                        for k in range(HIDDEN // 32):
                            a, b = plsc.unpack(
                                rowsf0[i, pl.ds(32 * k, 32)],
                                format=plsc.PackFormat.INTERLEAVED)
                            rowswb[i, pl.ds(32 * k, 16)] = a
                            rowswb[i, pl.ds(32 * k + 16, 16)] = b
                        return 0
                    lax.fori_loop(0, w, wb_body, 0)
                    pltpu.sync_copy(rowswb.at[pl.ds(0, w)],
                                    out.at[pl.ds(r0 + off, w)])
            plsc.subcore_barrier()
            return 0
        lax.fori_loop(0, nblk, blk_body, 0)

    return agg


_EDGE_DEFS = (
    ("seq", N_CHECKIN, E_PAD_BIG),
    ("vtd", N_CHECKIN, E_PAD_BIG),
    ("vis", N_POI, E_PAD_BIG),
    ("sp", N_POI, E_PAD_SP),
)


def _make_counts():
    mesh = plsc.VectorSubcoreMesh(core_axis_name="c", subcore_axis_name="s")
    allocs = {N_CHECKIN: 50176, N_POI: 10240}

    @functools.partial(
        pl.kernel,
        out_type=[jax.ShapeDtypeStruct((n, ), jnp.float32)
                  for _, n, _ in _EDGE_DEFS],
        mesh=mesh,
        compiler_params=pltpu.CompilerParams(needs_layout_passes=False),
        scratch_types=[
            pltpu.VMEM((C_EDGE,), jnp.int32),
            pltpu.VMEM((C_EDGE + 16,), jnp.int32),
            pltpu.VMEM((C_EDGE // G, G), jnp.int32),
            pltpu.VMEM((G,), jnp.float32),
            pltpu.VMEM((50176 // NSUB,), jnp.float32),
            pltpu.VMEM((1000,), jnp.float32),
            pltpu.VMEM_SHARED((50176,), jnp.float32),
            pltpu.VMEM_SHARED((50176,), jnp.float32),
            pltpu.VMEM_SHARED((10240,), jnp.float32),
            pltpu.VMEM_SHARED((10240,), jnp.float32),
        ],
    )
    def counts(d_seq, d_vtd, d_vis, d_sp,
               o_seq, o_vtd, o_vis, o_sp,
               dch, d1d, d2d, ones, zbuf, wbuf, c0, c1, c2, c3):
        cid = lax.axis_index("c")
        sid = lax.axis_index("s")
        one16 = jnp.ones((16,), jnp.float32)
        zero16f = jnp.zeros((16,), jnp.float32)

        def zo_body(i, _):
            ones[pl.ds(i * 16, 16)] = one16
            return 0
        lax.fori_loop(0, G // 16, zo_body, 0)

        def zz_body(i, _):
            zbuf[pl.ds(i * 16, 16)] = zero16f
            return 0
        lax.fori_loop(0, 50176 // NSUB // 16, zz_body, 0)

        for (nm, n_dst, e_pad), dst_in, out_ref, cspm in zip(
                _EDGE_DEFS, (d_seq, d_vtd, d_vis, d_sp),
                (o_seq, o_vtd, o_vis, o_sp), (c0, c1, c2, c3)):
            nhalf = n_dst // 2
            alloc = allocs[n_dst]
            share = alloc // NSUB
            et = e_pad // NSUB
            nch = et // C_EDGE
            lo = cid * nhalf
            dum16 = jnp.full((16,), nhalf, jnp.int32)

            pltpu.sync_copy(zbuf.at[pl.ds(0, share)],
                            cspm.at[pl.ds(sid * share, share)])
            plsc.subcore_barrier()

            def chunk_body(ch, _):
                base = sid * et + ch * C_EDGE
                pltpu.sync_copy(dst_in.at[pl.ds(base, C_EDGE)], dch)

                def zd_body(i, _):
                    d1d[pl.ds(i * 16, 16)] = dum16
                    return 0
                lax.fori_loop(0, C_EDGE // 16, zd_body, 0)

                def compact(i, cnt):
                    vd = dch[pl.ds(i * 16, 16)] - lo
                    m = jnp.logical_and(vd >= 0, vd < nhalf)
                    cum = plsc.cumsum(m.astype(jnp.int32))
                    pos = jnp.where(m, cum - 1 + cnt, C_EDGE)
                    plsc.store_scatter(d1d, [pos], vd)
                    return cnt + plsc.all_reduce_population_count(m)[0]
                cnt = lax.fori_loop(0, C_EDGE // 16, compact, jnp.int32(0))

                def fire(j, _):
                    @pl.when(j * G < cnt)
                    def _():
                        for k in range(G // 16):
                            d2d[j, pl.ds(k * 16, 16)] = \
                                d1d[pl.ds(j * G + k * 16, 16)]
                        pltpu.sync_copy(ones, cspm.at[d2d.at[j]], add=True)
                    return 0
                lax.fori_loop(0, C_EDGE // G, fire, 0)
                return 0
            lax.fori_loop(0, nch, chunk_body, 0)
            plsc.subcore_barrier()

            nwb = nhalf // 1000
            for k in range((nwb + NSUB - 1) // NSUB):
                idx = sid + k * NSUB

                @pl.when(idx < nwb)
                def _():
                    pltpu.sync_copy(cspm.at[pl.ds(idx * 1000, 1000)], wbuf)
                    pltpu.sync_copy(wbuf, out_ref.at[pl.ds(lo + idx * 1000,
                                                           1000)])
            plsc.subcore_barrier()

    return counts


_agg_cc = _make_agg(N_CHECKIN, N_CHECKIN, E_PAD_BIG)
_agg_pc = _make_agg(N_POI, N_CHECKIN, E_PAD_BIG)
_agg_cp = _make_agg(N_CHECKIN, N_POI, E_PAD_BIG)
_agg_pp = _make_agg(N_POI, N_POI, E_PAD_SP)
_counts_k = _make_counts()


def _transform_body(x_ref, W_ref, b_ref, deg_ref, h_ref, hbf_ref, hsbf_ref):
    h = jnp.dot(x_ref[:], W_ref[:], preferred_element_type=jnp.float32,
                precision=lax.Precision.HIGHEST) + b_ref[:]
    deg = deg_ref[:]
    dis = jnp.where(deg > 0.0, lax.rsqrt(jnp.maximum(deg, 1e-12)), 0.0)
    h_ref[:] = h
    hbf_ref[:] = h.astype(jnp.bfloat16)
    hsbf_ref[:] = (dis * h).astype(jnp.bfloat16)


def _transform(x, W, b, deg, n):
    row = pl.BlockSpec((BLK, HIDDEN), lambda i: (i, 0))
    return pl.pallas_call(
        _transform_body,
        grid=(n // BLK,),
        in_specs=[
            row,
            pl.BlockSpec((HIDDEN, HIDDEN), lambda i: (0, 0)),
            pl.BlockSpec((1, HIDDEN), lambda i: (0, 0)),
            pl.BlockSpec((BLK, 1), lambda i: (i, 0)),
        ],
        out_specs=[row, row, row],
        out_shape=[jax.ShapeDtypeStruct((n, HIDDEN), jnp.float32),
                   jax.ShapeDtypeStruct((n, HIDDEN), jnp.bfloat16),
                   jax.ShapeDtypeStruct((n, HIDDEN), jnp.bfloat16)],
    )(x, W, b.reshape(1, HIDDEN), deg)


def _combine_body(agg1_ref, agg2_ref, h_ref, deg_ref, cnt_ref,
                  W1_ref, W2_ref, W3_ref, b1_ref, b2_ref, pa_ref,
                  c_ref, csc_ref, *, with_prelu, with_scaled):
    deg = deg_ref[:]
    dis = jnp.where(deg > 0.0, lax.rsqrt(jnp.maximum(deg, 1e-12)), 0.0)
    invc = 1.0 / jnp.maximum(cnt_ref[:], 1.0)
    hi = lax.Precision.HIGHEST
    t = dis * jnp.dot(agg1_ref[:], W1_ref[:],
                      preferred_element_type=jnp.float32, precision=hi)
    t = t + b1_ref[:] + b2_ref[:]
    t = t + jnp.dot(invc * agg2_ref[:], W2_ref[:],
                    preferred_element_type=jnp.float32, precision=hi)
    t = t + jnp.dot(h_ref[:], W3_ref[:],
                    preferred_element_type=jnp.float32, precision=hi)
    if with_prelu:
        t = jnp.where(t >= 0.0, t, pa_ref[0, 0] * t)
    c_ref[:] = t
    if with_scaled:
        cbf_ref, csbf_ref = csc_ref
        cbf_ref[:] = t.astype(jnp.bfloat16)
        csbf_ref[:] = (dis * t).astype(jnp.bfloat16)


def _combine(agg1, agg2, h, deg, cnt, W1, W2, W3, b1, b2, pa, n,
             with_prelu, with_scaled):
    row = pl.BlockSpec((BLK, HIDDEN), lambda i: (i, 0))
    wspec = pl.BlockSpec((HIDDEN, HIDDEN), lambda i: (0, 0))
    bspec = pl.BlockSpec((1, HIDDEN), lambda i: (0, 0))
    col = pl.BlockSpec((BLK, 1), lambda i: (i, 0))
    body = functools.partial(_combine_body, with_prelu=with_prelu,
                             with_scaled=with_scaled)
    if with_scaled:
        def fn(a1, a2, hh, dg, ct, w1, w2, w3, bb1, bb2, paa, c, cbf, csbf):
            body(a1, a2, hh, dg, ct, w1, w2, w3, bb1, bb2, paa, c,
                 (cbf, csbf))
        nout = 3
        out_shape = [jax.ShapeDtypeStruct((n, HIDDEN), jnp.float32),
                     jax.ShapeDtypeStruct((n, HIDDEN), jnp.bfloat16),
                     jax.ShapeDtypeStruct((n, HIDDEN), jnp.bfloat16)]
    else:
        def fn(a1, a2, hh, dg, ct, w1, w2, w3, bb1, bb2, paa, c):
            body(a1, a2, hh, dg, ct, w1, w2, w3, bb1, bb2, paa, c, None)
        nout = 1
        out_shape = [jax.ShapeDtypeStruct((n, HIDDEN), jnp.float32)]
    out = pl.pallas_call(
        fn,
        grid=(n // BLK,),
        in_specs=[row, row, row, col, col, wspec, wspec, wspec, bspec, bspec,
                  pl.BlockSpec((1, 1), lambda i: (0, 0))],
        out_specs=[row] * nout,
        out_shape=out_shape,
    )(agg1, agg2, h, deg, cnt, W1, W2, W3,
      b1.reshape(1, HIDDEN), b2.reshape(1, HIDDEN), pa.reshape(1, 1))
    return out if with_scaled else (out[0], None, None)


def kernel(x_checkin, x_poi, ei_seq, ei_visits, ei_visited, ei_spatial,
           Wpc, bpc, Wpp, bpp, prelu_a,
           l1_seq_W, l1_seq_b, l1_vis_Wl, l1_vis_bl, l1_vis_Wr,
           l1_vtd_Wl, l1_vtd_bl, l1_vtd_Wr, l1_sp_W, l1_sp_b,
           l2_seq_W, l2_seq_b, l2_vis_Wl, l2_vis_bl, l2_vis_Wr,
           l2_vtd_Wl, l2_vtd_bl, l2_vtd_Wr, l2_sp_W, l2_sp_b):
    pa = jnp.asarray(prelu_a, jnp.float32)
    s_seq, d_seq = _pad_edges(ei_seq[0], ei_seq[1], N_CHECKIN, E_PAD_BIG)
    s_vtd, d_vtd = _pad_edges(ei_visited[0], ei_visited[1], N_CHECKIN, E_PAD_BIG)
    s_vis, d_vis = _pad_edges(ei_visits[0], ei_visits[1], N_POI, E_PAD_BIG)
    s_sp, d_sp = _pad_edges(ei_spatial[0], ei_spatial[1], N_POI, E_PAD_SP)

    deg_seq, cnt_vtd, cnt_vis, deg_sp = _counts_k(d_seq, d_vtd, d_vis, d_sp)
    deg_seq = deg_seq.reshape(N_CHECKIN, 1)
    cnt_vtd = cnt_vtd.reshape(N_CHECKIN, 1)
    cnt_vis = cnt_vis.reshape(N_POI, 1)
    deg_sp = deg_sp.reshape(N_POI, 1)

    perm = jnp.asarray(_PERM, jnp.int32)

    hc, hc_bf, hcs_bf = _transform(x_checkin, Wpc, bpc, deg_seq, N_CHECKIN)
    hp, hp_bf, hps_bf = _transform(x_poi, Wpp, bpp, deg_sp, N_POI)

    agg_seq = _agg_cc(_as_i32(hcs_bf), s_seq, d_seq)
    agg_vtd = _agg_pc(_as_i32(hp_bf), s_vtd, d_vtd)
    agg_vis = _agg_cp(_as_i32(hc_bf), s_vis, d_vis)
    agg_sp = _agg_pp(_as_i32(hps_bf), s_sp, d_sp)

    c1, c1_bf, c1s_bf = _combine(
        agg_seq, agg_vtd, hc, deg_seq, cnt_vtd,
        l1_seq_W[perm], l1_vtd_Wl[perm], l1_vtd_Wr, l1_seq_b, l1_vtd_bl,
        pa, N_CHECKIN, True, True)
    p1, p1_bf, p1s_bf = _combine(
        agg_sp, agg_vis, hp, deg_sp, cnt_vis,
        l1_sp_W[perm], l1_vis_Wl[perm], l1_vis_Wr, l1_sp_b, l1_vis_bl,
        pa, N_POI, True, True)

    agg_seq2 = _agg_cc(_as_i32(c1s_bf), s_seq, d_seq)
    agg_vtd2 = _agg_pc(_as_i32(p1_bf), s_vtd, d_vtd)
    agg_vis2 = _agg_cp(_as_i32(c1_bf), s_vis, d_vis)
    agg_sp2 = _agg_pp(_as_i32(p1s_bf), s_sp, d_sp)

    c2, _, _ = _combine(
        agg_seq2, agg_vtd2, c1, deg_seq, cnt_vtd,
        l2_seq_W[perm], l2_vtd_Wl[perm], l2_vtd_Wr, l2_seq_b, l2_vtd_bl,
        pa, N_CHECKIN, False, False)
    p2, _, _ = _combine(
        agg_sp2, agg_vis2, p1, deg_sp, cnt_vis,
        l2_sp_W[perm], l2_vis_Wl[perm], l2_vis_Wr, l2_sp_b, l2_vis_bl,
        pa, N_POI, False, False)
    return (c2, p2)
